# baseline (speedup 1.0000x reference)
import math

import jax
import jax.numpy as jnp
from jax import lax
from jax.experimental import pallas as pl
from jax.experimental.pallas import tpu as pltpu

_INV_SQRT2 = 1.0 / math.sqrt(2.0)
_MIB = 2 ** 20


def _round_up(a, b):
    return ((a + b - 1) // b) * b


def _cdiv(a, b):
    return (a + b - 1) // b


def _fused_mlp_kernel(x_ref, w1_ref, b1_ref, w2_ref, b2_ref, o_ref):
    k = pl.program_id(1)

    @pl.when(k == 0)
    def _init():
        o_ref[...] = jnp.broadcast_to(b2_ref[...], o_ref.shape)

    h = jnp.dot(x_ref[...], w1_ref[...].astype(jnp.bfloat16),
                preferred_element_type=jnp.float32)
    h = h + b1_ref[...]
    h = 0.5 * h * (1.0 + lax.erf(h * _INV_SQRT2))
    o_ref[...] += jnp.dot(h.astype(jnp.bfloat16),
                          w2_ref[...].astype(jnp.bfloat16),
                          preferred_element_type=jnp.float32)


def kernel(x, w1, b1, w2, b2):
    orig_shape = x.shape
    d_in = orig_shape[-1]
    d_hid = w1.shape[1]
    d_out = w2.shape[1]

    x2 = x.reshape(-1, d_in).astype(jnp.bfloat16)
    n_tok = x2.shape[0]

    if n_tok > 256:
        tm = _round_up(_cdiv(n_tok, 2), 16)
    else:
        tm = _round_up(max(n_tok, 16), 16)
    pad_m = (-n_tok) % tm
    if pad_m:
        x2 = jnp.pad(x2, ((0, pad_m), (0, 0)))
    n_pad = n_tok + pad_m

    tk = 256
    hid_pad = _round_up(d_hid, tk)
    if hid_pad != d_hid:
        w1 = jnp.pad(w1, ((0, 0), (0, hid_pad - d_hid)))
        b1 = jnp.pad(b1, ((0, hid_pad - d_hid),))
        w2 = jnp.pad(w2, ((0, hid_pad - d_hid), (0, 0)))

    b1_2d = b1.reshape(1, hid_pad).astype(jnp.float32)
    b2_2d = b2.reshape(1, d_out).astype(jnp.float32)

    grid = (n_pad // tm, hid_pad // tk)

    flops = 2 * n_pad * hid_pad * (d_in + d_out)
    bytes_accessed = (x2.size * 2
                      + w1.size * 4 * grid[0] + w2.size * 4 * grid[0]
                      + (hid_pad + d_out) * 4
                      + n_pad * d_out * 4)
    cost = pl.CostEstimate(flops=int(flops),
                           transcendentals=int(n_pad * hid_pad),
                           bytes_accessed=int(bytes_accessed))

    out = pl.pallas_call(
        _fused_mlp_kernel,
        out_shape=jax.ShapeDtypeStruct((n_pad, d_out), jnp.float32),
        grid=grid,
        in_specs=[
            pl.BlockSpec((tm, d_in), lambda i, k: (i, 0)),
            pl.BlockSpec((d_in, tk), lambda i, k: (0, k)),
            pl.BlockSpec((1, tk), lambda i, k: (0, k)),
            pl.BlockSpec((tk, d_out), lambda i, k: (k, 0)),
            pl.BlockSpec((1, d_out), lambda i, k: (0, 0)),
        ],
        out_specs=pl.BlockSpec((tm, d_out), lambda i, k: (i, 0)),
        compiler_params=pltpu.CompilerParams(
            dimension_semantics=("parallel", "arbitrary"),
            vmem_limit_bytes=62 * _MIB),
        cost_estimate=cost,
    )(x2, w1, b1_2d, w2, b2_2d)

    if pad_m:
        out = out[:n_tok]
    return out.reshape(orig_shape[:-1] + (d_out,)).astype(x.dtype)


# speedup vs baseline: 1.7180x; 1.7180x over previous
import functools
import math

import jax
import jax.numpy as jnp
from jax import lax
from jax.experimental import pallas as pl
from jax.experimental.pallas import tpu as pltpu

_INV_SQRT2 = 1.0 / math.sqrt(2.0)
_MIB = 2 ** 20
_SUB = 256


def _round_up(a, b):
    return ((a + b - 1) // b) * b


def _cdiv(a, b):
    return (a + b - 1) // b


def _fused_mlp_kernel(x_ref, w1_ref, b1_ref, w2_ref, b2_ref, o_ref, *, tk):
    k = pl.program_id(1)

    @pl.when(k == 0)
    def _init():
        o_ref[...] = jnp.broadcast_to(b2_ref[...], o_ref.shape)

    x = x_ref[...]
    hs = []
    for s in range(tk // _SUB):
        sl = pl.ds(s * _SUB, _SUB)
        h = jnp.dot(x, w1_ref[:, sl].astype(jnp.bfloat16),
                    preferred_element_type=jnp.float32)
        h = h + b1_ref[:, sl]
        h = 0.5 * h * (1.0 + lax.erf(h * _INV_SQRT2))
        hs.append(h.astype(jnp.bfloat16))
    h_all = hs[0] if len(hs) == 1 else jnp.concatenate(hs, axis=1)
    o_ref[...] += jnp.dot(h_all, w2_ref[...].astype(jnp.bfloat16),
                          preferred_element_type=jnp.float32)


def kernel(x, w1, b1, w2, b2):
    orig_shape = x.shape
    d_in = orig_shape[-1]
    d_hid = w1.shape[1]
    d_out = w2.shape[1]

    x2 = x.reshape(-1, d_in).astype(jnp.bfloat16)
    n_tok = x2.shape[0]

    tm = min(1024, _round_up(max(n_tok, 16), 16))
    pad_m = (-n_tok) % tm
    if pad_m:
        x2 = jnp.pad(x2, ((0, pad_m), (0, 0)))
    n_pad = n_tok + pad_m

    tk = 512
    hid_pad = _round_up(d_hid, tk)
    if hid_pad != d_hid:
        w1 = jnp.pad(w1, ((0, 0), (0, hid_pad - d_hid)))
        b1 = jnp.pad(b1, ((0, hid_pad - d_hid),))
        w2 = jnp.pad(w2, ((0, hid_pad - d_hid), (0, 0)))

    b1_2d = b1.reshape(1, hid_pad).astype(jnp.float32)
    b2_2d = b2.reshape(1, d_out).astype(jnp.float32)

    grid = (n_pad // tm, hid_pad // tk)

    flops = 2 * n_pad * hid_pad * (d_in + d_out)
    bytes_accessed = (x2.size * 2
                      + w1.size * 4 * grid[0] + w2.size * 4 * grid[0]
                      + (hid_pad + d_out) * 4
                      + n_pad * d_out * 4)
    cost = pl.CostEstimate(flops=int(flops),
                           transcendentals=int(n_pad * hid_pad),
                           bytes_accessed=int(bytes_accessed))

    out = pl.pallas_call(
        functools.partial(_fused_mlp_kernel, tk=tk),
        out_shape=jax.ShapeDtypeStruct((n_pad, d_out), jnp.float32),
        grid=grid,
        in_specs=[
            pl.BlockSpec((tm, d_in), lambda i, k: (i, 0)),
            pl.BlockSpec((d_in, tk), lambda i, k: (0, k)),
            pl.BlockSpec((1, tk), lambda i, k: (0, k)),
            pl.BlockSpec((tk, d_out), lambda i, k: (k, 0)),
            pl.BlockSpec((1, d_out), lambda i, k: (0, 0)),
        ],
        out_specs=pl.BlockSpec((tm, d_out), lambda i, k: (i, 0)),
        compiler_params=pltpu.CompilerParams(
            dimension_semantics=("arbitrary", "arbitrary"),
            vmem_limit_bytes=60 * _MIB),
        cost_estimate=cost,
    )(x2, w1, b1_2d, w2, b2_2d)

    if pad_m:
        out = out[:n_tok]
    return out.reshape(orig_shape[:-1] + (d_out,)).astype(x.dtype)


# speedup vs baseline: 1.7632x; 1.0263x over previous
import functools
import math

import jax
import jax.numpy as jnp
from jax import lax
from jax.experimental import pallas as pl
from jax.experimental.pallas import tpu as pltpu

_INV_SQRT2 = 1.0 / math.sqrt(2.0)
_MIB = 2 ** 20
_SUB = 256


def _round_up(a, b):
    return ((a + b - 1) // b) * b


def _cdiv(a, b):
    return (a + b - 1) // b


def _fused_mlp_kernel(x_ref, w1_ref, b1_ref, w2_ref, b2_ref, o_ref, *, tk):
    k = pl.program_id(1)

    @pl.when(k == 0)
    def _init():
        o_ref[...] = jnp.broadcast_to(b2_ref[...], o_ref.shape)

    x = x_ref[...]
    hs = []
    for s in range(tk // _SUB):
        sl = pl.ds(s * _SUB, _SUB)
        h = jnp.dot(x, w1_ref[:, sl].astype(jnp.bfloat16),
                    preferred_element_type=jnp.float32)
        h = h + b1_ref[:, sl]
        h = 0.5 * h * (1.0 + lax.erf(h * _INV_SQRT2))
        hs.append(h.astype(jnp.bfloat16))
    h_all = hs[0] if len(hs) == 1 else jnp.concatenate(hs, axis=1)
    o_ref[...] += jnp.dot(h_all, w2_ref[...].astype(jnp.bfloat16),
                          preferred_element_type=jnp.float32)


def kernel(x, w1, b1, w2, b2):
    orig_shape = x.shape
    d_in = orig_shape[-1]
    d_hid = w1.shape[1]
    d_out = w2.shape[1]

    x2 = x.reshape(-1, d_in).astype(jnp.bfloat16)
    n_tok = x2.shape[0]

    tm = min(1024, _round_up(max(n_tok, 16), 16))
    pad_m = (-n_tok) % tm
    if pad_m:
        x2 = jnp.pad(x2, ((0, pad_m), (0, 0)))
    n_pad = n_tok + pad_m

    tk = 1024
    hid_pad = _round_up(d_hid, tk)
    if hid_pad != d_hid:
        w1 = jnp.pad(w1, ((0, 0), (0, hid_pad - d_hid)))
        b1 = jnp.pad(b1, ((0, hid_pad - d_hid),))
        w2 = jnp.pad(w2, ((0, hid_pad - d_hid), (0, 0)))

    b1_2d = b1.reshape(1, hid_pad).astype(jnp.float32)
    b2_2d = b2.reshape(1, d_out).astype(jnp.float32)

    grid = (n_pad // tm, hid_pad // tk)

    flops = 2 * n_pad * hid_pad * (d_in + d_out)
    bytes_accessed = (x2.size * 2
                      + w1.size * 4 * grid[0] + w2.size * 4 * grid[0]
                      + (hid_pad + d_out) * 4
                      + n_pad * d_out * 4)
    cost = pl.CostEstimate(flops=int(flops),
                           transcendentals=int(n_pad * hid_pad),
                           bytes_accessed=int(bytes_accessed))

    out = pl.pallas_call(
        functools.partial(_fused_mlp_kernel, tk=tk),
        out_shape=jax.ShapeDtypeStruct((n_pad, d_out), jnp.float32),
        grid=grid,
        in_specs=[
            pl.BlockSpec((tm, d_in), lambda i, k: (i, 0)),
            pl.BlockSpec((d_in, tk), lambda i, k: (0, k)),
            pl.BlockSpec((1, tk), lambda i, k: (0, k)),
            pl.BlockSpec((tk, d_out), lambda i, k: (k, 0)),
            pl.BlockSpec((1, d_out), lambda i, k: (0, 0)),
        ],
        out_specs=pl.BlockSpec((tm, d_out), lambda i, k: (i, 0)),
        compiler_params=pltpu.CompilerParams(
            dimension_semantics=("arbitrary", "arbitrary"),
            vmem_limit_bytes=64 * _MIB),
        cost_estimate=cost,
    )(x2, w1, b1_2d, w2, b2_2d)

    if pad_m:
        out = out[:n_tok]
    return out.reshape(orig_shape[:-1] + (d_out,)).astype(x.dtype)
